# baseline (speedup 1.0000x reference)
import functools

import jax
import jax.numpy as jnp
from jax.experimental import pallas as pl
from jax.experimental.pallas import tpu as pltpu
from jax.experimental.pallas import tpu_sc as plsc

_K = 8
_BLK = 2048
_LANES = 128
_NSUB = _BLK // _LANES
_NFULL = 3
_M = 8
_CAP = 3


def _project(x, W, b):
    h = jax.lax.dot_general(x, W, (((1,), (1,)), ((), ())),
                            precision=jax.lax.Precision.DEFAULT,
                            preferred_element_type=jnp.float32)
    h = jnp.tanh(h + b)
    norm = jnp.maximum(jnp.sqrt(jnp.sum(h * h, axis=-1, keepdims=True)), 1e-08)
    scale = jnp.where(norm > 0.9, 0.9 / norm, jnp.ones_like(norm))
    return h * scale


def _merge8(topd, topi, m, ci, jidx):
    pos = jnp.sum((topd <= m).astype(jnp.int32), axis=1, keepdims=True)
    shifted_d = jnp.concatenate([topd[:, :1], topd[:, :_K - 1]], axis=1)
    shifted_i = jnp.concatenate([topi[:, :1], topi[:, :_K - 1]], axis=1)
    topd = jnp.where(jidx < pos, topd, jnp.where(jidx == pos, m, shifted_d))
    topi = jnp.where(jidx < pos, topi, jnp.where(jidx == pos, ci, shifted_i))
    return topd, topi


def _full_extract(dm, colidx, base, topd, topi, jidx):
    BIG = jnp.int32(2 ** 30)
    INF = jnp.float32(jnp.inf)
    for _ in range(_K):
        m = jnp.min(dm, axis=1, keepdims=True)
        am = jnp.min(jnp.where(dm == m, colidx, BIG), axis=1, keepdims=True)
        dm = jnp.where(colidx == am, INF, dm)
        topd, topi = _merge8(topd, topi, m, am + base, jidx)
    return topd, topi


def _common_block(i, q_ref, w_ref, b_ref, mem_ref, qp_ref, qsq_ref,
                  topd_ref, topi_ref, nreal):
    W = w_ref[...]
    bvec = b_ref[...]
    B = q_ref.shape[0]
    blk = mem_ref.shape[0]

    @pl.when(i == 0)
    def _init():
        qp = _project(q_ref[...], W, bvec)
        qp_ref[...] = qp
        qsq_ref[...] = jnp.sum(qp * qp, axis=-1, keepdims=True)
        topd_ref[...] = jnp.full((B, _K), jnp.inf, jnp.float32)
        topi_ref[...] = jnp.zeros((B, _K), jnp.int32)

    mp = _project(mem_ref[...], W, bvec)
    msq = jnp.sum(mp * mp, axis=-1, keepdims=True)
    prod = jax.lax.dot_general(qp_ref[...], mp, (((1,), (1,)), ((), ())),
                               precision=jax.lax.Precision.DEFAULT,
                               preferred_element_type=jnp.float32)
    sq = (qsq_ref[...] + msq.T) - 2.0 * prod
    d = jnp.sqrt(jnp.maximum(sq, 1e-12))

    base = i * blk
    colidx = jax.lax.broadcasted_iota(jnp.int32, (B, blk), 1)
    t8 = topd_ref[:, _K - 1:_K]
    c = (d < t8) & (colidx < nreal - base)
    dm = jnp.where(c, d, jnp.float32(jnp.inf))
    return c, dm, colidx, base


def _fin_step(i, nblk, topd_ref, topi_ref, wout_ref, iout_ref):
    @pl.when(i == nblk - 1)
    def _fin():
        td = topd_ref[...]
        wout_ref[...] = jax.nn.softmax((-td) / 0.1, axis=-1)
        iout_ref[...] = topi_ref[...]


def _body_full(nblk, nreal, q_ref, w_ref, b_ref, mem_ref, wout_ref, iout_ref,
               qp_ref, qsq_ref, topd_ref, topi_ref):
    i = pl.program_id(0)
    B = q_ref.shape[0]
    jidx = jax.lax.broadcasted_iota(jnp.int32, (B, _K), 1)
    c, dm, colidx, base = _common_block(
        i, q_ref, w_ref, b_ref, mem_ref, qp_ref, qsq_ref, topd_ref, topi_ref,
        nreal)
    topd, topi = _full_extract(dm, colidx, base, topd_ref[...], topi_ref[...],
                               jidx)
    topd_ref[...] = topd
    topi_ref[...] = topi
    _fin_step(i, nblk, topd_ref, topi_ref, wout_ref, iout_ref)


def _body_hybrid(nblk, nreal, q_ref, w_ref, b_ref, mem_ref, wout_ref,
                 iout_ref, sus_ref, qp_ref, qsq_ref, topd_ref, topi_ref,
                 gval_ref, gidx_ref, cacc_ref, susacc_ref):
    i = pl.program_id(0)
    B = q_ref.shape[0]
    INF = jnp.float32(jnp.inf)
    BIG = jnp.int32(2 ** 30)
    jidx = jax.lax.broadcasted_iota(jnp.int32, (B, _K), 1)

    @pl.when(i == 0)
    def _init_state():
        susacc_ref[...] = jnp.zeros((B, 1), jnp.float32)
        cacc_ref[...] = jnp.zeros((B, _LANES), jnp.float32)
        gval_ref[...] = jnp.full((B, _K * _LANES), INF, jnp.float32)
        gidx_ref[...] = jnp.full((B, _K * _LANES), BIG, jnp.int32)

    c, dm, colidx, base = _common_block(
        i, q_ref, w_ref, b_ref, mem_ref, qp_ref, qsq_ref, topd_ref, topi_ref,
        nreal)

    @pl.when(i < _NFULL)
    def _full():
        topd, topi = _full_extract(dm, colidx, base, topd_ref[...],
                                   topi_ref[...], jidx)
        topd_ref[...] = topd
        topi_ref[...] = topi

    @pl.when(i >= _NFULL)
    def _fast():
        lane = jax.lax.broadcasted_iota(jnp.int32, (B, _LANES), 1)
        hs = [jnp.full((B, _LANES), INF, jnp.float32) for _ in range(_CAP)]
        his = [jnp.full((B, _LANES), BIG, jnp.int32) for _ in range(_CAP)]
        cnt = jnp.zeros((B, _LANES), jnp.float32)
        for r in range(_NSUB):
            v = dm[:, r * _LANES:(r + 1) * _LANES]
            vi = lane + (base + r * _LANES)
            cnt = cnt + c[:, r * _LANES:(r + 1) * _LANES].astype(jnp.float32)
            for j in range(_CAP):
                lt = v < hs[j]
                nv = jnp.where(lt, hs[j], v)
                nvi = jnp.where(lt, his[j], vi)
                hs[j] = jnp.where(lt, v, hs[j])
                his[j] = jnp.where(lt, vi, his[j])
                v, vi = nv, nvi
        cacc = cacc_ref[...] + cnt
        cacc_ref[...] = cacc
        bad = jnp.maximum(jnp.max(cnt, axis=1, keepdims=True) - (_CAP + 0.5),
                          jnp.max(cacc, axis=1, keepdims=True) - (_K + 0.5))
        susacc_ref[...] = jnp.maximum(susacc_ref[...],
                                      jnp.where(bad > 0.0, 1.0, 0.0))
        g = [gval_ref[:, j * _LANES:(j + 1) * _LANES] for j in range(_K)]
        gi = [gidx_ref[:, j * _LANES:(j + 1) * _LANES] for j in range(_K)]
        for jc in range(_CAP):
            v, vi = hs[jc], his[jc]
            for j in range(_K):
                lt = v < g[j]
                nv = jnp.where(lt, g[j], v)
                nvi = jnp.where(lt, gi[j], vi)
                g[j] = jnp.where(lt, v, g[j])
                gi[j] = jnp.where(lt, vi, gi[j])
                v, vi = nv, nvi
        for j in range(_K):
            gval_ref[:, j * _LANES:(j + 1) * _LANES] = g[j]
            gidx_ref[:, j * _LANES:(j + 1) * _LANES] = gi[j]

    is_drain = jnp.logical_or(
        jnp.logical_and(i >= _NFULL, (i + 1) % _M == 0), i == nblk - 1)

    @pl.when(is_drain)
    def _drain():
        w = [gval_ref[:, j * _LANES:(j + 1) * _LANES] for j in range(_K)]
        wi = [gidx_ref[:, j * _LANES:(j + 1) * _LANES] for j in range(_K)]
        topd = topd_ref[...]
        topi = topi_ref[...]
        for _ in range(_K):
            m = jnp.min(w[0], axis=1, keepdims=True)
            ei = jnp.min(jnp.where(w[0] == m, wi[0], BIG), axis=1,
                         keepdims=True)
            pro = wi[0] == ei
            for j in range(_K - 1):
                w[j] = jnp.where(pro, w[j + 1], w[j])
                wi[j] = jnp.where(pro, wi[j + 1], wi[j])
            w[_K - 1] = jnp.where(pro, INF, w[_K - 1])
            wi[_K - 1] = jnp.where(pro, BIG, wi[_K - 1])
            topd, topi = _merge8(topd, topi, m, ei, jidx)
        topd_ref[...] = topd
        topi_ref[...] = topi
        gval_ref[...] = jnp.full((B, _K * _LANES), INF, jnp.float32)
        gidx_ref[...] = jnp.full((B, _K * _LANES), BIG, jnp.int32)
        cacc_ref[...] = jnp.zeros((B, _LANES), jnp.float32)

    @pl.when(i == nblk - 1)
    def _fin_sus():
        sus_ref[...] = susacc_ref[...]

    _fin_step(i, nblk, topd_ref, topi_ref, wout_ref, iout_ref)


def _topk_call(query, memory_embeddings, W, b2, mode, interpret=False):
    B, D = query.shape
    N = memory_embeddings.shape[0]
    nblk = (N + _BLK - 1) // _BLK
    hybrid = mode == "hybrid"
    body = _body_hybrid if hybrid else _body_full
    out_specs = [
        pl.BlockSpec((B, _K), lambda i: (0, 0)),
        pl.BlockSpec((B, _K), lambda i: (0, 0)),
    ]
    out_shape = [
        jax.ShapeDtypeStruct((B, _K), jnp.float32),
        jax.ShapeDtypeStruct((B, _K), jnp.int32),
    ]
    scratch = [
        pltpu.VMEM((B, D), jnp.float32),
        pltpu.VMEM((B, 1), jnp.float32),
        pltpu.VMEM((B, _K), jnp.float32),
        pltpu.VMEM((B, _K), jnp.int32),
    ]
    if hybrid:
        out_specs.append(pl.BlockSpec((B, 1), lambda i: (0, 0)))
        out_shape.append(jax.ShapeDtypeStruct((B, 1), jnp.float32))
        scratch.extend([
            pltpu.VMEM((B, _K * _LANES), jnp.float32),
            pltpu.VMEM((B, _K * _LANES), jnp.int32),
            pltpu.VMEM((B, _LANES), jnp.float32),
            pltpu.VMEM((B, 1), jnp.float32),
        ])
    out = pl.pallas_call(
        functools.partial(body, nblk, N),
        grid=(nblk,),
        in_specs=[
            pl.BlockSpec((B, D), lambda i: (0, 0)),
            pl.BlockSpec((D, D), lambda i: (0, 0)),
            pl.BlockSpec((1, D), lambda i: (0, 0)),
            pl.BlockSpec((_BLK, D), lambda i: (i, 0)),
        ],
        out_specs=out_specs,
        out_shape=out_shape,
        scratch_shapes=scratch,
        interpret=interpret,
    )(query, W, b2, memory_embeddings)
    return out


def _gather_outcomes(memory_outcomes, flat_idx):
    num_indices = flat_idx.shape[1]
    value_dim = memory_outcomes.shape[1]
    window = 128
    mesh = plsc.VectorSubcoreMesh(core_axis_name="core",
                                  subcore_axis_name="subcore")

    @pl.kernel(out_type=jax.ShapeDtypeStruct((num_indices, value_dim),
                                             memory_outcomes.dtype),
               mesh=mesh)
    def kern(x_hbm, i_hbm, o_hbm):
        def body(i_vmem, o_vmem):
            pltpu.sync_copy(x_hbm.at[i_vmem.at[0]], o_vmem)

        pltpu.emit_pipeline(
            body,
            grid=(num_indices // window,),
            in_specs=[pl.BlockSpec((1, window), index_map=lambda i: (0, i))],
            out_specs=[pl.BlockSpec((window, value_dim),
                                    index_map=lambda i: (i, 0))],
            core_axis_name="subcore",
            dimension_semantics=(pltpu.PARALLEL,),
        )(i_hbm, o_hbm)

    return kern(memory_outcomes, flat_idx)


def kernel(query, memory_embeddings, memory_outcomes, W, b, k):
    B, D = query.shape
    b2 = jnp.reshape(b, (1, D)).astype(jnp.float32)
    weights, idx, suspect = _topk_call(query, memory_embeddings, W, b2,
                                       "hybrid")
    bad = jnp.max(suspect) > 0.0

    def _redo(_):
        w, ix = _topk_call(query, memory_embeddings, W, b2, "full")
        return w, ix

    def _keep(_):
        return weights, idx

    weights, idx = jax.lax.cond(bad, _redo, _keep, None)
    flat_idx = idx.reshape(1, B * _K)
    paired = memory_outcomes.reshape(-1, 2 * D)
    gathered = _gather_outcomes(paired, flat_idx // 2)
    halves = gathered.reshape(B, _K, 2, D)
    odd = (idx % 2 == 1)[..., None]
    outcomes = jnp.where(odd, halves[:, :, 1, :], halves[:, :, 0, :])
    return weights, outcomes


# speedup vs baseline: 2.5838x; 2.5838x over previous
import functools

import jax
import jax.numpy as jnp
from jax.experimental import pallas as pl
from jax.experimental.pallas import tpu as pltpu
from jax.experimental.pallas import tpu_sc as plsc

_K = 8
_BLK = 2048
_LANES = 128
_NSUB = _BLK // _LANES
_M = 8
_CAP = 4


def _project(x, W, b):
    h = jax.lax.dot_general(x, W, (((1,), (1,)), ((), ())),
                            precision=jax.lax.Precision.DEFAULT,
                            preferred_element_type=jnp.float32)
    h = jnp.tanh(h + b)
    norm = jnp.maximum(jnp.sqrt(jnp.sum(h * h, axis=-1, keepdims=True)), 1e-08)
    scale = jnp.where(norm > 0.9, 0.9 / norm, jnp.ones_like(norm))
    return h * scale


def _merge8(topd, topi, m, ci, jidx):
    pos = jnp.sum((topd <= m).astype(jnp.int32), axis=1, keepdims=True)
    shifted_d = jnp.concatenate([topd[:, :1], topd[:, :_K - 1]], axis=1)
    shifted_i = jnp.concatenate([topi[:, :1], topi[:, :_K - 1]], axis=1)
    topd = jnp.where(jidx < pos, topd, jnp.where(jidx == pos, m, shifted_d))
    topi = jnp.where(jidx < pos, topi, jnp.where(jidx == pos, ci, shifted_i))
    return topd, topi


def _full_extract(dm, colidx, base, topd, topi, jidx):
    BIG = jnp.int32(2 ** 30)
    INF = jnp.float32(jnp.inf)
    for _ in range(_K):
        m = jnp.min(dm, axis=1, keepdims=True)
        am = jnp.min(jnp.where(dm == m, colidx, BIG), axis=1, keepdims=True)
        dm = jnp.where(colidx == am, INF, dm)
        topd, topi = _merge8(topd, topi, m, am + base, jidx)
    return topd, topi


def _dists(qp, qsq, mem, W, bvec):
    mp = _project(mem, W, bvec)
    msq = jnp.sum(mp * mp, axis=-1, keepdims=True)
    prod = jax.lax.dot_general(qp, mp, (((1,), (1,)), ((), ())),
                               precision=jax.lax.Precision.DEFAULT,
                               preferred_element_type=jnp.float32)
    sq = (qsq + msq.T) - 2.0 * prod
    return jnp.sqrt(jnp.maximum(sq, 1e-12))


def _body_seed(q_ref, w_ref, b_ref, mem_ref, qp_ref, qsq_ref, topd_ref,
               topi_ref):
    B = q_ref.shape[0]
    blk = mem_ref.shape[0]
    W = w_ref[...]
    bvec = b_ref[...]
    qp = _project(q_ref[...], W, bvec)
    qp_ref[...] = qp
    qsq = jnp.sum(qp * qp, axis=-1, keepdims=True)
    qsq_ref[...] = qsq
    d = _dists(qp, qsq, mem_ref[...], W, bvec)
    colidx = jax.lax.broadcasted_iota(jnp.int32, (B, blk), 1)
    jidx = jax.lax.broadcasted_iota(jnp.int32, (B, _K), 1)
    topd = jnp.full((B, _K), jnp.inf, jnp.float32)
    topi = jnp.zeros((B, _K), jnp.int32)
    topd, topi = _full_extract(d, colidx, 0, topd, topi, jidx)
    topd_ref[...] = topd
    topi_ref[...] = topi


def _body_main(nreal, qp_ref, qsq_ref, topd0_ref, topi0_ref, w_ref, b_ref,
               mem_ref, wout_ref, iout_ref, sus_ref, topd_ref, topi_ref,
               gval_ref, gidx_ref, cacc_ref, susacc_ref):
    o = pl.program_id(0)
    t = pl.program_id(1)
    B = qp_ref.shape[0]
    blk = mem_ref.shape[0]
    INF = jnp.float32(jnp.inf)
    BIG = jnp.int32(2 ** 30)
    first = jnp.logical_and(o == 0, t == 0)

    @pl.when(first)
    def _init():
        topd_ref[...] = topd0_ref[...]
        topi_ref[...] = topi0_ref[...]
        susacc_ref[...] = jnp.zeros((B, 1), jnp.float32)

    @pl.when(t == 0)
    def _reset():
        gval_ref[...] = jnp.full((B, _K * _LANES), INF, jnp.float32)
        gidx_ref[...] = jnp.full((B, _K * _LANES), BIG, jnp.int32)
        cacc_ref[...] = jnp.zeros((B, _LANES), jnp.float32)

    bi = 1 + o * _M + t
    base = bi * blk
    d = _dists(qp_ref[...], qsq_ref[...], mem_ref[...], w_ref[...], b_ref[...])
    colidx = jax.lax.broadcasted_iota(jnp.int32, (B, blk), 1)
    t8 = topd_ref[:, _K - 1:_K]
    c = (d < t8) & (colidx < nreal - base)
    dm = jnp.where(c, d, INF)

    lane = jax.lax.broadcasted_iota(jnp.int32, (B, _LANES), 1)
    hs = [jnp.full((B, _LANES), INF, jnp.float32) for _ in range(_CAP)]
    his = [jnp.full((B, _LANES), BIG, jnp.int32) for _ in range(_CAP)]
    cnt = jnp.zeros((B, _LANES), jnp.float32)
    for r in range(_NSUB):
        v = dm[:, r * _LANES:(r + 1) * _LANES]
        vi = lane + (base + r * _LANES)
        cnt = cnt + c[:, r * _LANES:(r + 1) * _LANES].astype(jnp.float32)
        for j in range(_CAP):
            lt = v < hs[j]
            nv = jnp.where(lt, hs[j], v)
            nvi = jnp.where(lt, his[j], vi)
            hs[j] = jnp.where(lt, v, hs[j])
            his[j] = jnp.where(lt, vi, his[j])
            v, vi = nv, nvi
    cacc = cacc_ref[...] + cnt
    cacc_ref[...] = cacc
    bad = jnp.maximum(jnp.max(cnt, axis=1, keepdims=True) - (_CAP + 0.5),
                      jnp.max(cacc, axis=1, keepdims=True) - (_K + 0.5))
    susacc_ref[...] = jnp.maximum(susacc_ref[...],
                                  jnp.where(bad > 0.0, 1.0, 0.0))
    g = [gval_ref[:, j * _LANES:(j + 1) * _LANES] for j in range(_K)]
    gi = [gidx_ref[:, j * _LANES:(j + 1) * _LANES] for j in range(_K)]
    for jc in range(_CAP):
        v, vi = hs[jc], his[jc]
        for j in range(_K):
            lt = v < g[j]
            nv = jnp.where(lt, g[j], v)
            nvi = jnp.where(lt, gi[j], vi)
            g[j] = jnp.where(lt, v, g[j])
            gi[j] = jnp.where(lt, vi, gi[j])
            v, vi = nv, nvi
    for j in range(_K):
        gval_ref[:, j * _LANES:(j + 1) * _LANES] = g[j]
        gidx_ref[:, j * _LANES:(j + 1) * _LANES] = gi[j]

    @pl.when(t == _M - 1)
    def _drain():
        jidx = jax.lax.broadcasted_iota(jnp.int32, (B, _K), 1)
        w = [gval_ref[:, j * _LANES:(j + 1) * _LANES] for j in range(_K)]
        wi = [gidx_ref[:, j * _LANES:(j + 1) * _LANES] for j in range(_K)]
        topd = topd_ref[...]
        topi = topi_ref[...]
        for _ in range(_K):
            m = jnp.min(w[0], axis=1, keepdims=True)
            ei = jnp.min(jnp.where(w[0] == m, wi[0], BIG), axis=1,
                         keepdims=True)
            pro = wi[0] == ei
            for j in range(_K - 1):
                w[j] = jnp.where(pro, w[j + 1], w[j])
                wi[j] = jnp.where(pro, wi[j + 1], wi[j])
            w[_K - 1] = jnp.where(pro, INF, w[_K - 1])
            wi[_K - 1] = jnp.where(pro, BIG, wi[_K - 1])
            topd, topi = _merge8(topd, topi, m, ei, jidx)
        topd_ref[...] = topd
        topi_ref[...] = topi
        wout_ref[...] = jax.nn.softmax((-topd) / 0.1, axis=-1)
        iout_ref[...] = topi
        sus_ref[...] = susacc_ref[...]


def _body_full(nblk, nreal, q_ref, w_ref, b_ref, mem_ref, wout_ref, iout_ref,
               qp_ref, qsq_ref, topd_ref, topi_ref):
    i = pl.program_id(0)
    B = q_ref.shape[0]
    blk = mem_ref.shape[0]
    W = w_ref[...]
    bvec = b_ref[...]
    jidx = jax.lax.broadcasted_iota(jnp.int32, (B, _K), 1)

    @pl.when(i == 0)
    def _init():
        qp = _project(q_ref[...], W, bvec)
        qp_ref[...] = qp
        qsq_ref[...] = jnp.sum(qp * qp, axis=-1, keepdims=True)
        topd_ref[...] = jnp.full((B, _K), jnp.inf, jnp.float32)
        topi_ref[...] = jnp.zeros((B, _K), jnp.int32)

    d = _dists(qp_ref[...], qsq_ref[...], mem_ref[...], W, bvec)
    base = i * blk
    colidx = jax.lax.broadcasted_iota(jnp.int32, (B, blk), 1)
    t8 = topd_ref[:, _K - 1:_K]
    c = (d < t8) & (colidx < nreal - base)
    dm = jnp.where(c, d, jnp.float32(jnp.inf))
    topd, topi = _full_extract(dm, colidx, base, topd_ref[...], topi_ref[...],
                               jidx)
    topd_ref[...] = topd
    topi_ref[...] = topi

    @pl.when(i == nblk - 1)
    def _fin():
        wout_ref[...] = jax.nn.softmax((-topd_ref[...]) / 0.1, axis=-1)
        iout_ref[...] = topi_ref[...]


def _topk_fast(query, memory_embeddings, W, b2, interpret=False):
    B, D = query.shape
    N = memory_embeddings.shape[0]
    qp, qsq, topd0, topi0 = pl.pallas_call(
        _body_seed,
        grid=(1,),
        in_specs=[
            pl.BlockSpec((B, D), lambda i: (0, 0)),
            pl.BlockSpec((D, D), lambda i: (0, 0)),
            pl.BlockSpec((1, D), lambda i: (0, 0)),
            pl.BlockSpec((_BLK, D), lambda i: (0, 0)),
        ],
        out_specs=[
            pl.BlockSpec((B, D), lambda i: (0, 0)),
            pl.BlockSpec((B, 1), lambda i: (0, 0)),
            pl.BlockSpec((B, _K), lambda i: (0, 0)),
            pl.BlockSpec((B, _K), lambda i: (0, 0)),
        ],
        out_shape=[
            jax.ShapeDtypeStruct((B, D), jnp.float32),
            jax.ShapeDtypeStruct((B, 1), jnp.float32),
            jax.ShapeDtypeStruct((B, _K), jnp.float32),
            jax.ShapeDtypeStruct((B, _K), jnp.int32),
        ],
        interpret=interpret,
    )(query, W, b2, memory_embeddings)

    nper = (N + _BLK - 1) // _BLK - 1
    nout = nper // _M
    assert nout * _M == nper
    weights, idx, sus = pl.pallas_call(
        functools.partial(_body_main, N),
        grid=(nout, _M),
        in_specs=[
            pl.BlockSpec((B, D), lambda o, t: (0, 0)),
            pl.BlockSpec((B, 1), lambda o, t: (0, 0)),
            pl.BlockSpec((B, _K), lambda o, t: (0, 0)),
            pl.BlockSpec((B, _K), lambda o, t: (0, 0)),
            pl.BlockSpec((D, D), lambda o, t: (0, 0)),
            pl.BlockSpec((1, D), lambda o, t: (0, 0)),
            pl.BlockSpec((_BLK, D), lambda o, t: (1 + o * _M + t, 0)),
        ],
        out_specs=[
            pl.BlockSpec((B, _K), lambda o, t: (0, 0)),
            pl.BlockSpec((B, _K), lambda o, t: (0, 0)),
            pl.BlockSpec((B, 1), lambda o, t: (0, 0)),
        ],
        out_shape=[
            jax.ShapeDtypeStruct((B, _K), jnp.float32),
            jax.ShapeDtypeStruct((B, _K), jnp.int32),
            jax.ShapeDtypeStruct((B, 1), jnp.float32),
        ],
        scratch_shapes=[
            pltpu.VMEM((B, _K), jnp.float32),
            pltpu.VMEM((B, _K), jnp.int32),
            pltpu.VMEM((B, _K * _LANES), jnp.float32),
            pltpu.VMEM((B, _K * _LANES), jnp.int32),
            pltpu.VMEM((B, _LANES), jnp.float32),
            pltpu.VMEM((B, 1), jnp.float32),
        ],
        interpret=interpret,
    )(qp, qsq, topd0, topi0, W, b2, memory_embeddings)
    return weights, idx, sus


def _topk_full(query, memory_embeddings, W, b2, interpret=False):
    B, D = query.shape
    N = memory_embeddings.shape[0]
    nblk = (N + _BLK - 1) // _BLK
    return pl.pallas_call(
        functools.partial(_body_full, nblk, N),
        grid=(nblk,),
        in_specs=[
            pl.BlockSpec((B, D), lambda i: (0, 0)),
            pl.BlockSpec((D, D), lambda i: (0, 0)),
            pl.BlockSpec((1, D), lambda i: (0, 0)),
            pl.BlockSpec((_BLK, D), lambda i: (i, 0)),
        ],
        out_specs=[
            pl.BlockSpec((B, _K), lambda i: (0, 0)),
            pl.BlockSpec((B, _K), lambda i: (0, 0)),
        ],
        out_shape=[
            jax.ShapeDtypeStruct((B, _K), jnp.float32),
            jax.ShapeDtypeStruct((B, _K), jnp.int32),
        ],
        scratch_shapes=[
            pltpu.VMEM((B, D), jnp.float32),
            pltpu.VMEM((B, 1), jnp.float32),
            pltpu.VMEM((B, _K), jnp.float32),
            pltpu.VMEM((B, _K), jnp.int32),
        ],
        interpret=interpret,
    )(query, W, b2, memory_embeddings)


def _gather_outcomes(memory_outcomes, flat_idx):
    num_indices = flat_idx.shape[1]
    value_dim = memory_outcomes.shape[1]
    window = 128
    mesh = plsc.VectorSubcoreMesh(core_axis_name="core",
                                  subcore_axis_name="subcore")

    @pl.kernel(out_type=jax.ShapeDtypeStruct((num_indices, value_dim),
                                             memory_outcomes.dtype),
               mesh=mesh)
    def kern(x_hbm, i_hbm, o_hbm):
        def body(i_vmem, o_vmem):
            pltpu.sync_copy(x_hbm.at[i_vmem.at[0]], o_vmem)

        pltpu.emit_pipeline(
            body,
            grid=(num_indices // window,),
            in_specs=[pl.BlockSpec((1, window), index_map=lambda i: (0, i))],
            out_specs=[pl.BlockSpec((window, value_dim),
                                    index_map=lambda i: (i, 0))],
            core_axis_name="subcore",
            dimension_semantics=(pltpu.PARALLEL,),
        )(i_hbm, o_hbm)

    return kern(memory_outcomes, flat_idx)


def kernel(query, memory_embeddings, memory_outcomes, W, b, k):
    B, D = query.shape
    b2 = jnp.reshape(b, (1, D)).astype(jnp.float32)
    weights, idx, suspect = _topk_fast(query, memory_embeddings, W, b2)
    bad = jnp.max(suspect) > 0.0

    def _redo(_):
        return _topk_full(query, memory_embeddings, W, b2)

    def _keep(_):
        return weights, idx

    weights, idx = jax.lax.cond(bad, _redo, _keep, None)
    flat_idx = idx.reshape(1, B * _K)
    paired = memory_outcomes.reshape(-1, 2 * D)
    gathered = _gather_outcomes(paired, flat_idx // 2)
    halves = gathered.reshape(B, _K, 2, D)
    odd = (idx % 2 == 1)[..., None]
    outcomes = jnp.where(odd, halves[:, :, 1, :], halves[:, :, 0, :])
    return weights, outcomes
